# Initial kernel scaffold; baseline (speedup 1.0000x reference)
import jax
import jax.numpy as jnp
from jax.experimental import pallas as pl


def kernel(start_nodes, current_times, memory_state, dense_neighbor_ids, dense_neighbor_times, dense_neighbor_counts, gumbel_u, restart_u, time_freq, time_phase, W_restart, b_restart):
    raise NotImplementedError("write your pallas kernel here")


# speedup vs baseline: 2.2878x; 2.2878x over previous
import functools

import jax
import jax.numpy as jnp
from jax import lax
from jax.experimental import pallas as pl
from jax.experimental.pallas import tpu as pltpu
from jax.experimental.pallas import tpu_sc as plsc

NUM_NODES = 100000
DEG = 32
BATCH = 4096
NWALKS = 5
WLEN = 8
MDIM = 128
TDIM = 64
TEMP = 0.1
N = BATCH * NWALKS
NC, NS, LANES = 2, 16, 16
NTILES = NC * NS
PW = N // NTILES
CHUNK = 128
NCHUNK = PW // CHUNK
GRID = 8192
GPAD = 65 * 128
ROWB = 2000


def _tmask_body(t_ref, cnt_ref, o_ref):
    slot = lax.broadcasted_iota(jnp.int32, t_ref.shape, 1)
    o_ref[...] = jnp.where(slot < cnt_ref[...], t_ref[...], jnp.float32(2.0))


_tmask = pl.pallas_call(
    _tmask_body,
    grid=(NUM_NODES // ROWB,),
    in_specs=[
        pl.BlockSpec((ROWB, DEG), lambda i: (i, 0)),
        pl.BlockSpec((ROWB, 1), lambda i: (i, 0)),
    ],
    out_specs=pl.BlockSpec((ROWB, DEG), lambda i: (i, 0)),
    out_shape=jax.ShapeDtypeStruct((NUM_NODES, DEG), jnp.float32),
)


def _memdot_body(m_ref, w_ref, b_ref, o_ref):
    o_ref[...] = (
        jnp.dot(m_ref[...], w_ref[...], preferred_element_type=jnp.float32)
        + b_ref[0, 0]
    )


_memdot = pl.pallas_call(
    _memdot_body,
    grid=(NUM_NODES // ROWB,),
    in_specs=[
        pl.BlockSpec((ROWB, MDIM), lambda i: (i, 0)),
        pl.BlockSpec((MDIM, 1), lambda i: (0, 0)),
        pl.BlockSpec((1, 1), lambda i: (0, 0)),
    ],
    out_specs=pl.BlockSpec((ROWB, 1), lambda i: (i, 0)),
    out_shape=jax.ShapeDtypeStruct((NUM_NODES, 1), jnp.float32),
)


def _tlgrid_body(f_ref, p_ref, w_ref, o_ref):
    i0 = lax.broadcasted_iota(jnp.int32, (GPAD // 128, 128), 0)
    i1 = lax.broadcasted_iota(jnp.int32, (GPAD // 128, 128), 1)
    t = (i0 * 128 + i1).astype(jnp.float32) * jnp.float32(1.0 / GRID)
    acc = jnp.zeros((GPAD // 128, 128), jnp.float32)
    for k in range(TDIM):
        acc = acc + jnp.cos(t * f_ref[k // 8, k % 8] + p_ref[k // 8, k % 8]) * w_ref[k // 8, k % 8]
    o_ref[...] = acc


_tlgrid = pl.pallas_call(
    _tlgrid_body,
    out_shape=jax.ShapeDtypeStruct((GPAD // 128, 128), jnp.float32),
)


def _gumbel_body(u_ref, o_ref):
    u = jnp.clip(u_ref[...], jnp.float32(1e-6), jnp.float32(1.0 - 1e-6))
    g = -jnp.log(-jnp.log(u))
    o_ref[...] = g.reshape(g.shape[0], WLEN, DEG).transpose(1, 0, 2)


_gumbel = pl.pallas_call(
    _gumbel_body,
    grid=(10,),
    in_specs=[pl.BlockSpec((N // 10, WLEN * DEG), lambda i: (i, 0))],
    out_specs=pl.BlockSpec((WLEN, N // 10, DEG), lambda i: (0, i, 0)),
    out_shape=jax.ShapeDtypeStruct((WLEN, N, DEG), jnp.float32),
)


def _walk_body(
    tm_h, ids_h, md_h, tlg_h, gt_h, ru_h, st_h, sn2_h,
    walks_h, rho_h,
    curt_v, md_v, curidx_v, rows_t, rows_i, gst_v, ru_v,
    sn_v, st_v, tlg_v, wout_v, rout_v,
    sem, sem2,
):
    wid = lax.axis_index("s") * NC + lax.axis_index("c")
    base = wid * PW
    iota16 = lax.broadcasted_iota(jnp.int32, (LANES,), 0)

    pltpu.sync_copy(sn2_h.at[wid], sn_v)
    pltpu.sync_copy(st_h.at[pl.ds(base, PW)], st_v)
    pltpu.sync_copy(st_h.at[pl.ds(base, PW)], curt_v)
    pltpu.sync_copy(sn2_h.at[wid], curidx_v)
    pltpu.sync_copy(ru_h.at[pl.ds(base, PW)], ru_v)
    pltpu.sync_copy(tlg_h, tlg_v)
    mcs = [
        pltpu.async_copy(md_h.at[curidx_v.at[k]], md_v.at[pl.ds(k * CHUNK, CHUNK)], sem2)
        for k in range(NCHUNK)
    ]
    for c in mcs:
        c.wait()

    def _step(s, carry):
        row_cps = []
        for k in range(NCHUNK):
            row_cps.append(pltpu.async_copy(tm_h.at[curidx_v.at[k]], rows_t.at[k], sem))
            row_cps.append(pltpu.async_copy(ids_h.at[curidx_v.at[k]], rows_i.at[k], sem))
        pltpu.sync_copy(gt_h.at[s, pl.ds(base, PW)], gst_v)

        def _rho(g2, c):
            off = g2 * LANES
            curt16 = curt_v[pl.ds(off, LANES)]
            md16 = md_v[pl.ds(off, LANES)]
            x = curt16 * jnp.float32(GRID)
            ji = x.astype(jnp.int32)
            ji = jnp.minimum(jnp.maximum(ji, 0), GRID - 1)
            fr = x - ji.astype(jnp.float32)
            jip = ji + 1
            t0 = plsc.load_gather(tlg_v, [ji >> 7, ji & 127])
            t1 = plsc.load_gather(tlg_v, [jip >> 7, jip & 127])
            logit = md16 + (t0 + fr * (t1 - t0))
            rho16 = 1.0 / (1.0 + jnp.exp(-logit))
            sv = jnp.broadcast_to(s, (LANES,))
            plsc.store_scatter(rout_v, [off + iota16, sv], rho16)
            return c

        lax.fori_loop(0, PW // LANES, _rho, 0)
        for c in row_cps:
            c.wait()

        def _choice(g2, c):
            off = g2 * LANES
            chunk = g2 // (CHUNK // LANES)
            chunkv = jnp.broadcast_to(chunk, (LANES,))
            row16 = (g2 % (CHUNK // LANES)) * LANES + iota16
            widx = off + iota16
            curt16 = curt_v[pl.ds(off, LANES)]
            cur16 = plsc.load_gather(curidx_v, [chunkv, row16])
            neginf = jnp.full((LANES,), -jnp.inf, jnp.float32)
            tmax = neginf
            tjs = []
            for j in range(DEG):
                tj = plsc.load_gather(rows_t, [chunkv, row16, jnp.full((LANES,), j, jnp.int32)])
                tjs.append(tj)
                tmax = jnp.maximum(tmax, jnp.where(tj < curt16, tj, neginf))
            hasv = tmax > jnp.float32(-1e30)
            tmax_safe = jnp.where(hasv, tmax, jnp.float32(0.0))
            best = neginf
            bid = jnp.full((LANES,), 0, jnp.int32)
            bt = curt16
            for j in range(DEG):
                tj = tjs[j]
                jv = jnp.full((LANES,), j, jnp.int32)
                sc_ = jnp.where(tj < curt16, (tj - tmax_safe) / jnp.float32(TEMP),
                                jnp.float32(-1e9))
                score = sc_ + plsc.load_gather(gst_v, [widx, jv])
                upd = score > best
                best = jnp.where(upd, score, best)
                bid = jnp.where(upd, jv, bid)
                bt = jnp.where(upd, tj, bt)
            nid = plsc.load_gather(rows_i, [chunkv, row16, bid])
            nid = jnp.where(hasv, nid, cur16)
            nt = jnp.where(hasv, bt, curt16)
            sv = jnp.broadcast_to(s, (LANES,))
            rho16 = plsc.load_gather(rout_v, [widx, sv])
            ru16 = plsc.load_gather(ru_v, [widx, sv])
            restart = ru16 < rho16
            sn16 = plsc.load_gather(sn_v, [chunkv, row16])
            newc = jnp.where(restart, sn16, nid)
            newt = jnp.where(restart, st_v[pl.ds(off, LANES)], nt)
            curt_v[pl.ds(off, LANES)] = newt
            plsc.store_scatter(curidx_v, [chunkv, row16], newc)
            plsc.store_scatter(wout_v, [widx, sv], newc)
            return c

        lax.fori_loop(0, PW // LANES, _choice, 0)
        mcs2 = [
            pltpu.async_copy(md_h.at[curidx_v.at[k]], md_v.at[pl.ds(k * CHUNK, CHUNK)], sem2)
            for k in range(NCHUNK)
        ]
        for c in mcs2:
            c.wait()
        return carry

    lax.fori_loop(0, WLEN, _step, 0)
    pltpu.sync_copy(wout_v, walks_h.at[wid])
    pltpu.sync_copy(rout_v, rho_h.at[wid])


_walk = functools.partial(
    pl.kernel,
    out_type=[
        jax.ShapeDtypeStruct((NTILES, PW, WLEN), jnp.int32),
        jax.ShapeDtypeStruct((NTILES, PW, WLEN), jnp.float32),
    ],
    mesh=plsc.VectorSubcoreMesh(
        core_axis_name="c", subcore_axis_name="s", num_cores=NC, num_subcores=NS
    ),
    compiler_params=pltpu.CompilerParams(
        needs_layout_passes=False, use_tc_tiling_on_sc=False
    ),
    scratch_types=[
        pltpu.VMEM((PW,), jnp.float32),
        pltpu.VMEM((PW,), jnp.float32),
        pltpu.VMEM((NCHUNK, CHUNK), jnp.int32),
        pltpu.VMEM((NCHUNK, CHUNK, DEG), jnp.float32),
        pltpu.VMEM((NCHUNK, CHUNK, DEG), jnp.int32),
        pltpu.VMEM((PW, DEG), jnp.float32),
        pltpu.VMEM((PW, WLEN), jnp.float32),
        pltpu.VMEM((NCHUNK, CHUNK), jnp.int32),
        pltpu.VMEM((PW,), jnp.float32),
        pltpu.VMEM((GPAD // 128, 128), jnp.float32),
        pltpu.VMEM((PW, WLEN), jnp.int32),
        pltpu.VMEM((PW, WLEN), jnp.float32),
        pltpu.SemaphoreType.DMA,
        pltpu.SemaphoreType.DMA,
    ],
)(_walk_body)


def kernel(start_nodes, current_times, memory_state, dense_neighbor_ids,
           dense_neighbor_times, dense_neighbor_counts, gumbel_u, restart_u,
           time_freq, time_phase, W_restart, b_restart):
    tm = _tmask(dense_neighbor_times, dense_neighbor_counts.reshape(NUM_NODES, 1))
    md = _memdot(
        memory_state,
        W_restart[0, :MDIM].reshape(MDIM, 1),
        b_restart.reshape(1, 1),
    ).reshape(NUM_NODES)
    tlg = _tlgrid(
        time_freq.reshape(8, 8), time_phase.reshape(8, 8),
        W_restart[0, MDIM:].reshape(8, 8),
    )
    gt = _gumbel(gumbel_u.reshape(N, WLEN * DEG))
    ru = restart_u.reshape(N, WLEN)
    sn = jnp.repeat(start_nodes, NWALKS)
    st = jnp.repeat(current_times, NWALKS)
    sn2 = sn.reshape(NTILES, NCHUNK, CHUNK)
    walks_f, rho_f = _walk(tm, dense_neighbor_ids, md, tlg, gt, ru, st, sn2)
    return (
        walks_f.reshape(BATCH, NWALKS, WLEN),
        rho_f.reshape(BATCH, NWALKS, WLEN),
    )


# speedup vs baseline: 2.9575x; 1.2927x over previous
import functools

import jax
import jax.numpy as jnp
from jax import lax
from jax.experimental import pallas as pl
from jax.experimental.pallas import tpu as pltpu
from jax.experimental.pallas import tpu_sc as plsc

NUM_NODES = 100000
DEG = 32
BATCH = 4096
NWALKS = 5
WLEN = 8
MDIM = 128
TDIM = 64
N = BATCH * NWALKS
NC, NS, LANES = 2, 16, 16
NTILES = NC * NS
PW = N // NTILES
CHUNK = 128
NCHUNK = PW // CHUNK
GRID = 8192
GPAD = 65 * 128
ROWB = 2000


def _prep_body(t_ref, cnt_ref, m_ref, w_ref, b_ref, tm_ref, md_ref):
    slot = lax.broadcasted_iota(jnp.int32, t_ref.shape, 1)
    tm_ref[...] = jnp.where(slot < cnt_ref[...], t_ref[...], jnp.float32(2.0))
    md_ref[...] = (
        jnp.dot(m_ref[...], w_ref[...], preferred_element_type=jnp.float32)
        + b_ref[0, 0]
    )


_prep = pl.pallas_call(
    _prep_body,
    grid=(NUM_NODES // ROWB,),
    in_specs=[
        pl.BlockSpec((ROWB, DEG), lambda i: (i, 0)),
        pl.BlockSpec((ROWB, 1), lambda i: (i, 0)),
        pl.BlockSpec((ROWB, MDIM), lambda i: (i, 0)),
        pl.BlockSpec((MDIM, 1), lambda i: (0, 0)),
        pl.BlockSpec((1, 1), lambda i: (0, 0)),
    ],
    out_specs=[
        pl.BlockSpec((ROWB, DEG), lambda i: (i, 0)),
        pl.BlockSpec((ROWB, 1), lambda i: (i, 0)),
    ],
    out_shape=[
        jax.ShapeDtypeStruct((NUM_NODES, DEG), jnp.float32),
        jax.ShapeDtypeStruct((NUM_NODES, 1), jnp.float32),
    ],
)


def _tlgrid_body(f_ref, p_ref, w_ref, o_ref):
    i0 = lax.broadcasted_iota(jnp.int32, (GPAD // 128, 128), 0)
    i1 = lax.broadcasted_iota(jnp.int32, (GPAD // 128, 128), 1)
    t = (i0 * 128 + i1).astype(jnp.float32) * jnp.float32(1.0 / GRID)
    acc = jnp.zeros((GPAD // 128, 128), jnp.float32)
    for k in range(TDIM):
        acc = acc + jnp.cos(t * f_ref[k // 8, k % 8] + p_ref[k // 8, k % 8]) * w_ref[k // 8, k % 8]
    o_ref[...] = acc


_tlgrid = pl.pallas_call(
    _tlgrid_body,
    out_shape=jax.ShapeDtypeStruct((GPAD // 128, 128), jnp.float32),
)


def _gumbel_body(u_ref, o_ref):
    u = jnp.clip(u_ref[...], jnp.float32(1e-6), jnp.float32(1.0 - 1e-6))
    o_ref[...] = -jnp.log(-jnp.log(u))


_gumbel = pl.pallas_call(
    _gumbel_body,
    grid=(10,),
    in_specs=[pl.BlockSpec((WLEN, DEG, N // 10), lambda i: (0, 0, i))],
    out_specs=pl.BlockSpec((WLEN, DEG, N // 10), lambda i: (0, 0, i)),
    out_shape=jax.ShapeDtypeStruct((WLEN, DEG, N), jnp.float32),
)


def _walk_body(
    tm_h, ids_h, md_h, tlg_h, gt_h, ru_h, st_h, sn2_h,
    walks_h, rho_h,
    curt_v, md_v, curidx_v, rows_t, rows_i, gst_v, ru_v,
    sn_v, st_v, tlg_v, wout_v, rout_v,
    sem, sem2,
):
    wid = lax.axis_index("s") * NC + lax.axis_index("c")
    base = wid * PW
    iota16 = lax.broadcasted_iota(jnp.int32, (LANES,), 0)

    pltpu.sync_copy(sn2_h.at[wid], sn_v)
    pltpu.sync_copy(st_h.at[pl.ds(base, PW)], st_v)
    pltpu.sync_copy(st_h.at[pl.ds(base, PW)], curt_v)
    pltpu.sync_copy(sn2_h.at[wid], curidx_v)
    pltpu.sync_copy(ru_h.at[:, pl.ds(base, PW)], ru_v)
    pltpu.sync_copy(tlg_h, tlg_v)
    mcs = [
        pltpu.async_copy(
            md_h.at[curidx_v.at[pl.ds(k * CHUNK, CHUNK)]],
            md_v.at[pl.ds(k * CHUNK, CHUNK)], sem2)
        for k in range(NCHUNK)
    ]

    for s in range(WLEN):
        row_cps = []
        for k in range(NCHUNK):
            idx = curidx_v.at[pl.ds(k * CHUNK, CHUNK)]
            dst = pl.ds(k * CHUNK, CHUNK)
            row_cps.append(pltpu.async_copy(tm_h.at[idx], rows_t.at[dst], sem))
            row_cps.append(pltpu.async_copy(ids_h.at[idx], rows_i.at[dst], sem))
        pltpu.sync_copy(gt_h.at[s, :, pl.ds(base, PW)], gst_v)
        for c in mcs:
            c.wait()

        def _rho(g2, c, s=s):
            off = g2 * LANES
            curt16 = curt_v[pl.ds(off, LANES)]
            md16 = md_v[pl.ds(off, LANES)]
            x = curt16 * jnp.float32(GRID)
            ji = x.astype(jnp.int32)
            fr = x - ji.astype(jnp.float32)
            t0 = plsc.load_gather(tlg_v, [ji])
            t1 = plsc.load_gather(tlg_v, [ji + 1])
            logit = md16 + (t0 + fr * (t1 - t0))
            rho16 = 1.0 / (1.0 + jnp.exp(-logit))
            rout_v[s, pl.ds(off, LANES)] = rho16
            return c

        lax.fori_loop(0, PW // LANES, _rho, 0)
        for c in row_cps:
            c.wait()

        def _choice(g2, c, s=s):
            off = g2 * LANES
            widx = off + iota16
            curt16 = curt_v[pl.ds(off, LANES)]
            neg = jnp.full((LANES,), -1e9, jnp.float32)
            best = jnp.full((LANES,), -jnp.inf, jnp.float32)
            bid = jnp.full((LANES,), 0, jnp.int32)
            hasv = jnp.full((LANES,), False, jnp.bool_)
            for j in range(DEG):
                jv = jnp.full((LANES,), j, jnp.int32)
                tj = plsc.load_gather(rows_t, [widx, jv])
                c1 = tj < curt16
                hasv = hasv | c1
                score = jnp.where(c1, tj * jnp.float32(10.0), neg)
                score = score + gst_v[j, pl.ds(off, LANES)]
                upd = score > best
                best = jnp.where(upd, score, best)
                bid = jnp.where(upd, jv, bid)
            nt = plsc.load_gather(rows_t, [widx, bid])
            nid = plsc.load_gather(rows_i, [widx, bid])
            cur16 = curidx_v[pl.ds(off, LANES)]
            nid = jnp.where(hasv, nid, cur16)
            nt = jnp.where(hasv, nt, curt16)
            rho16 = rout_v[s, pl.ds(off, LANES)]
            ru16 = ru_v[s, pl.ds(off, LANES)]
            restart = ru16 < rho16
            newc = jnp.where(restart, sn_v[pl.ds(off, LANES)], nid)
            newt = jnp.where(restart, st_v[pl.ds(off, LANES)], nt)
            curt_v[pl.ds(off, LANES)] = newt
            curidx_v[pl.ds(off, LANES)] = newc
            wout_v[s, pl.ds(off, LANES)] = newc
            return c

        lax.fori_loop(0, PW // LANES, _choice, 0)
        if s + 1 < WLEN:
            mcs = [
                pltpu.async_copy(
                    md_h.at[curidx_v.at[pl.ds(k * CHUNK, CHUNK)]],
                    md_v.at[pl.ds(k * CHUNK, CHUNK)], sem2)
                for k in range(NCHUNK)
            ]

    pltpu.sync_copy(wout_v, walks_h.at[wid])
    pltpu.sync_copy(rout_v, rho_h.at[wid])


_walk = functools.partial(
    pl.kernel,
    out_type=[
        jax.ShapeDtypeStruct((NTILES, WLEN, PW), jnp.int32),
        jax.ShapeDtypeStruct((NTILES, WLEN, PW), jnp.float32),
    ],
    mesh=plsc.VectorSubcoreMesh(
        core_axis_name="c", subcore_axis_name="s", num_cores=NC, num_subcores=NS
    ),
    compiler_params=pltpu.CompilerParams(
        needs_layout_passes=False, use_tc_tiling_on_sc=False
    ),
    scratch_types=[
        pltpu.VMEM((PW,), jnp.float32),
        pltpu.VMEM((PW,), jnp.float32),
        pltpu.VMEM((PW,), jnp.int32),
        pltpu.VMEM((PW, DEG), jnp.float32),
        pltpu.VMEM((PW, DEG), jnp.int32),
        pltpu.VMEM((DEG, PW), jnp.float32),
        pltpu.VMEM((WLEN, PW), jnp.float32),
        pltpu.VMEM((PW,), jnp.int32),
        pltpu.VMEM((PW,), jnp.float32),
        pltpu.VMEM((GPAD,), jnp.float32),
        pltpu.VMEM((WLEN, PW), jnp.int32),
        pltpu.VMEM((WLEN, PW), jnp.float32),
        pltpu.SemaphoreType.DMA,
        pltpu.SemaphoreType.DMA,
    ],
)(_walk_body)


def kernel(start_nodes, current_times, memory_state, dense_neighbor_ids,
           dense_neighbor_times, dense_neighbor_counts, gumbel_u, restart_u,
           time_freq, time_phase, W_restart, b_restart):
    tm, md = _prep(
        dense_neighbor_times,
        dense_neighbor_counts.reshape(NUM_NODES, 1),
        memory_state,
        W_restart[0, :MDIM].reshape(MDIM, 1),
        b_restart.reshape(1, 1),
    )
    md = md.reshape(NUM_NODES)
    tlg = _tlgrid(
        time_freq.reshape(8, 8), time_phase.reshape(8, 8),
        W_restart[0, MDIM:].reshape(8, 8),
    ).reshape(GPAD)
    gu_t = jnp.transpose(gumbel_u.reshape(N, WLEN, DEG), (1, 2, 0))
    gt = _gumbel(gu_t)
    ru = jnp.transpose(restart_u.reshape(N, WLEN))
    sn = jnp.repeat(start_nodes, NWALKS)
    st = jnp.repeat(current_times, NWALKS)
    sn2 = sn.reshape(NTILES, PW)
    walks_f, rho_f = _walk(tm, dense_neighbor_ids, md, tlg, gt, ru, st, sn2)
    return (
        jnp.transpose(walks_f, (0, 2, 1)).reshape(BATCH, NWALKS, WLEN),
        jnp.transpose(rho_f, (0, 2, 1)).reshape(BATCH, NWALKS, WLEN),
    )
